# Initial kernel scaffold; baseline (speedup 1.0000x reference)
import jax
import jax.numpy as jnp
from jax.experimental import pallas as pl


def kernel(s0, ss, z, embedding):
    raise NotImplementedError("write your pallas kernel here")


# speedup vs baseline: 1.6833x; 1.6833x over previous
import functools

import jax
import jax.numpy as jnp
from jax import lax
from jax.experimental import pallas as pl
from jax.experimental.pallas import tpu as pltpu
from jax.experimental.pallas import tpu_sc as plsc

_EMBED_DIM = 64
_NUM_CORES = 2
_NUM_SUBCORES = 16
_NUM_WORKERS = _NUM_CORES * _NUM_SUBCORES
_CHUNK = 128


@functools.lru_cache(maxsize=None)
def _make_gather(n_rows):
    n_per_w = n_rows // _NUM_WORKERS
    n_chunks = n_per_w // _CHUNK
    mesh = plsc.VectorSubcoreMesh(core_axis_name="c", subcore_axis_name="s")

    def body(table_hbm, idx_hbm, out_hbm, idx_v, rows_v, gsem):
        wid = lax.axis_index("s") * _NUM_CORES + lax.axis_index("c")
        pltpu.sync_copy(idx_hbm.at[pl.ds(wid * n_chunks, n_chunks)], idx_v)
        row_base = wid * n_per_w

        def step(j, carry):
            pltpu.async_copy(table_hbm.at[idx_v.at[j]], rows_v, gsem).wait()
            pltpu.sync_copy(
                rows_v, out_hbm.at[pl.ds(row_base + j * _CHUNK, _CHUNK)]
            )
            return carry

        lax.fori_loop(0, n_chunks, step, 0)

    return pl.kernel(
        body,
        mesh=mesh,
        out_type=jax.ShapeDtypeStruct((n_rows, _EMBED_DIM), jnp.float32),
        scratch_types=[
            pltpu.VMEM((n_chunks, _CHUNK), jnp.int32),
            pltpu.VMEM((_CHUNK, _EMBED_DIM), jnp.float32),
            pltpu.SemaphoreType.DMA,
        ],
        compiler_params=pltpu.CompilerParams(use_tc_tiling_on_sc=False),
    )


def kernel(s0, ss, z, embedding):
    b, h = z.shape
    n_rows = b * h
    idx = z.astype(jnp.int32).reshape(n_rows // _CHUNK, _CHUNK)
    out = _make_gather(n_rows)(embedding, idx)
    return out.reshape(b, h, _EMBED_DIM)


# speedup vs baseline: 1.8273x; 1.0856x over previous
import functools

import jax
import jax.numpy as jnp
from jax import lax
from jax.experimental import pallas as pl
from jax.experimental.pallas import tpu as pltpu
from jax.experimental.pallas import tpu_sc as plsc

_EMBED_DIM = 64
_NUM_CORES = 2
_NUM_SUBCORES = 16
_NUM_WORKERS = _NUM_CORES * _NUM_SUBCORES
_CHUNK = 512


@functools.lru_cache(maxsize=None)
def _make_gather(n_rows):
    n_per_w = n_rows // _NUM_WORKERS
    n_chunks = n_per_w // _CHUNK
    mesh = plsc.VectorSubcoreMesh(core_axis_name="c", subcore_axis_name="s")

    def body(table_hbm, idx_hbm, out_hbm, idx_v, rows_v, gsem):
        wid = lax.axis_index("s") * _NUM_CORES + lax.axis_index("c")
        pltpu.sync_copy(idx_hbm.at[pl.ds(wid * n_chunks, n_chunks)], idx_v)
        row_base = wid * n_per_w

        def step(j, carry):
            pltpu.async_copy(table_hbm.at[idx_v.at[j]], rows_v, gsem).wait()
            pltpu.sync_copy(
                rows_v, out_hbm.at[pl.ds(row_base + j * _CHUNK, _CHUNK)]
            )
            return carry

        lax.fori_loop(0, n_chunks, step, 0)

    return pl.kernel(
        body,
        mesh=mesh,
        out_type=jax.ShapeDtypeStruct((n_rows, _EMBED_DIM), jnp.float32),
        scratch_types=[
            pltpu.VMEM((n_chunks, _CHUNK), jnp.int32),
            pltpu.VMEM((_CHUNK, _EMBED_DIM), jnp.float32),
            pltpu.SemaphoreType.DMA,
        ],
        compiler_params=pltpu.CompilerParams(use_tc_tiling_on_sc=False),
    )


def kernel(s0, ss, z, embedding):
    b, h = z.shape
    n_rows = b * h
    idx = z.astype(jnp.int32).reshape(n_rows // _CHUNK, _CHUNK)
    out = _make_gather(n_rows)(embedding, idx)
    return out.reshape(b, h, _EMBED_DIM)


# speedup vs baseline: 1.8736x; 1.0254x over previous
import functools

import jax
import jax.numpy as jnp
from jax import lax
from jax.experimental import pallas as pl
from jax.experimental.pallas import tpu as pltpu
from jax.experimental.pallas import tpu_sc as plsc

_EMBED_DIM = 64
_NUM_CORES = 2
_NUM_SUBCORES = 16
_NUM_WORKERS = _NUM_CORES * _NUM_SUBCORES
_CHUNK = 512
_NBUF = 3


@functools.lru_cache(maxsize=None)
def _make_gather(n_rows):
    n_per_w = n_rows // _NUM_WORKERS
    n_chunks = n_per_w // _CHUNK
    mesh = plsc.VectorSubcoreMesh(core_axis_name="c", subcore_axis_name="s")

    def body(table_hbm, idx_hbm, out_hbm, idx_v, rows_v, gsem, osem):
        wid = lax.axis_index("s") * _NUM_CORES + lax.axis_index("c")
        pltpu.sync_copy(idx_hbm.at[pl.ds(wid * n_chunks, n_chunks)], idx_v)
        row_base = wid * n_per_w

        def gather(chunk, slot):
            return pltpu.make_async_copy(
                table_hbm.at[idx_v.at[chunk]], rows_v.at[slot], gsem
            )

        def out_copy(chunk, slot):
            return pltpu.make_async_copy(
                rows_v.at[slot],
                out_hbm.at[pl.ds(row_base + chunk * _CHUNK, _CHUNK)],
                osem,
            )

        for s in range(_NBUF - 1):
            gather(s, s).start()

        def step(j, carry):
            prev_slot = (j + _NBUF - 1) % _NBUF

            @pl.when(j > 0)
            def _():
                out_copy(j - 1, prev_slot).wait()

            @pl.when(j + _NBUF - 1 < n_chunks)
            def _():
                gather(j + _NBUF - 1, prev_slot).start()

            slot = j % _NBUF
            gather(j, slot).wait()
            out_copy(j, slot).start()
            return carry

        lax.fori_loop(0, n_chunks, step, 0)
        out_copy(n_chunks - 1, (n_chunks - 1) % _NBUF).wait()

    return pl.kernel(
        body,
        mesh=mesh,
        out_type=jax.ShapeDtypeStruct((n_rows, _EMBED_DIM), jnp.float32),
        scratch_types=[
            pltpu.VMEM((n_chunks, _CHUNK), jnp.int32),
            pltpu.VMEM((_NBUF, _CHUNK, _EMBED_DIM), jnp.float32),
            pltpu.SemaphoreType.DMA,
            pltpu.SemaphoreType.DMA,
        ],
        compiler_params=pltpu.CompilerParams(use_tc_tiling_on_sc=False),
    )


def kernel(s0, ss, z, embedding):
    b, h = z.shape
    n_rows = b * h
    idx = z.astype(jnp.int32).reshape(n_rows // _CHUNK, _CHUNK)
    out = _make_gather(n_rows)(embedding, idx)
    return out.reshape(b, h, _EMBED_DIM)


# speedup vs baseline: 1.8740x; 1.0002x over previous
import functools

import jax
import jax.numpy as jnp
from jax import lax
from jax.experimental import pallas as pl
from jax.experimental.pallas import tpu as pltpu
from jax.experimental.pallas import tpu_sc as plsc

_EMBED_DIM = 64
_NUM_CORES = 2
_NUM_SUBCORES = 16
_NUM_WORKERS = _NUM_CORES * _NUM_SUBCORES
_BPC = 2
_NBUF = 4


@functools.lru_cache(maxsize=None)
def _make_gather(batch, hist):
    rows_per_chunk = _BPC * hist
    b_per_w = batch // _NUM_WORKERS
    n_chunks = b_per_w // _BPC
    mesh = plsc.VectorSubcoreMesh(core_axis_name="c", subcore_axis_name="s")

    def body(table_hbm, idx_hbm, out_hbm, idx_v, rows_v, gsem, osem):
        wid = lax.axis_index("s") * _NUM_CORES + lax.axis_index("c")
        pltpu.sync_copy(idx_hbm.at[pl.ds(wid * n_chunks, n_chunks)], idx_v)
        batch_base = wid * b_per_w

        def gather(chunk, slot):
            return pltpu.make_async_copy(
                table_hbm.at[idx_v.at[chunk]], rows_v.at[slot], gsem
            )

        def out_copies(chunk, slot):
            b0 = batch_base + chunk * _BPC
            return [
                pltpu.make_async_copy(
                    rows_v.at[slot, pl.ds(i * hist, hist)],
                    out_hbm.at[b0 + i],
                    osem,
                )
                for i in range(_BPC)
            ]

        for s in range(_NBUF - 1):
            gather(s, s).start()

        def step(j, carry):
            prev_slot = (j + _NBUF - 1) % _NBUF

            @pl.when(j > 0)
            def _():
                for c in out_copies(j - 1, prev_slot):
                    c.wait()

            @pl.when(j + _NBUF - 1 < n_chunks)
            def _():
                gather(j + _NBUF - 1, prev_slot).start()

            slot = j % _NBUF
            gather(j, slot).wait()
            for c in out_copies(j, slot):
                c.start()
            return carry

        lax.fori_loop(0, n_chunks, step, 0)
        for c in out_copies(n_chunks - 1, (n_chunks - 1) % _NBUF):
            c.wait()

    return pl.kernel(
        body,
        mesh=mesh,
        out_type=jax.ShapeDtypeStruct((batch, hist, _EMBED_DIM), jnp.float32),
        scratch_types=[
            pltpu.VMEM((n_chunks, rows_per_chunk), jnp.int32),
            pltpu.VMEM((_NBUF, rows_per_chunk, _EMBED_DIM), jnp.float32),
            pltpu.SemaphoreType.DMA,
            pltpu.SemaphoreType.DMA,
        ],
        compiler_params=pltpu.CompilerParams(use_tc_tiling_on_sc=False),
    )


def kernel(s0, ss, z, embedding):
    b, h = z.shape
    idx = z.astype(jnp.int32).reshape(b * h // (_BPC * h), _BPC * h)
    return _make_gather(b, h)(embedding, idx)
